# baseline (speedup 1.0000x reference)
import functools

import jax
import jax.numpy as jnp
from jax import lax
from jax.experimental import pallas as pl
from jax.experimental.pallas import tpu as pltpu
from jax.experimental.pallas import tpu_sc as plsc

NC = 2
NS = 16
LANES = 16
CHUNK = 128


def _make_sc_agg(n_table, d, n_pad, steps0, steps1):
    steps_max = max(steps0, steps1)
    rpt = n_pad // NS
    mesh = plsc.VectorSubcoreMesh(core_axis_name="c", subcore_axis_name="s")

    @functools.partial(
        pl.kernel,
        mesh=mesh,
        compiler_params=pltpu.CompilerParams(use_tc_tiling_on_sc=False),
        out_type=jax.ShapeDtypeStruct((NC, n_pad, d), jnp.float32),
        scratch_types=[
            pltpu.VMEM((steps_max, 2, CHUNK), jnp.int32),
            pltpu.VMEM((CHUNK, d), jnp.float32),
            pltpu.VMEM_SHARED((n_pad, d), jnp.float32),
            pltpu.SemaphoreType.DMA,
        ],
    )
    def agg(table_hbm, edges_hbm, zeros_hbm, out_hbm,
            idx_v, rows_v, acc_sh, sem):
        c = lax.axis_index("c")
        s = lax.axis_index("s")
        tile = c * NS + s
        steps = jnp.where(c == 0, steps0, steps1)
        pltpu.sync_copy(zeros_hbm, acc_sh.at[pl.ds(s * rpt, rpt)])
        pltpu.sync_copy(edges_hbm.at[tile], idx_v)
        plsc.subcore_barrier()

        def body(i, carry):
            pltpu.async_copy(table_hbm.at[idx_v.at[i, 0]], rows_v, sem).wait()
            pltpu.sync_copy(rows_v, acc_sh.at[idx_v.at[i, 1]], add=True)
            return carry

        lax.fori_loop(0, steps, body, 0)
        plsc.subcore_barrier()
        pltpu.sync_copy(acc_sh.at[pl.ds(s * rpt, rpt)],
                        out_hbm.at[c, pl.ds(s * rpt, rpt)])

    return agg


def _tc1_body(x_ref, agg_ref, w1a_ref, b1a_ref, w1b_ref, b1b_ref, w2a_ref,
              out_ref):
    z = x_ref[...] + agg_ref[0] + agg_ref[1]
    z1 = jnp.dot(z, w1a_ref[...], preferred_element_type=jnp.float32)
    z1 = jnp.maximum(z1 + b1a_ref[...], 0.0)
    h1 = jnp.dot(z1, w1b_ref[...], preferred_element_type=jnp.float32)
    h1 = h1 + b1b_ref[...]
    out_ref[...] = jnp.dot(h1, w2a_ref[...], preferred_element_type=jnp.float32)


def _tc2_body(g2_ref, agg_ref, b2a_ref, w2b_ref, b2b_ref, out_ref):
    z = g2_ref[...] + agg_ref[0] + agg_ref[1]
    z = jnp.maximum(z + b2a_ref[...], 0.0)
    h2 = jnp.dot(z, w2b_ref[...], preferred_element_type=jnp.float32)
    h2 = h2 + b2b_ref[...]
    m = jnp.max(h2, axis=1, keepdims=True)
    lse = jnp.log(jnp.sum(jnp.exp(h2 - m), axis=1, keepdims=True)) + m
    out_ref[...] = h2 - lse


def kernel(x, edge_index, W1a, b1a, W1b, b1b, W2a, b2a, W2b, b2b):
    n, f = x.shape
    e = edge_index.shape[1]
    c_out = W2b.shape[1]
    d2 = 48

    n_tiles = NC * NS
    blk = 1024
    grid = (-(-n // blk),)
    rpt = -(-(n + 1 + CHUNK) // (NS * 8)) * 8
    n_pad = rpt * NS

    tot = -(-e // (NS * CHUNK))

    def split_steps(share0):
        s0 = -(-max(4, min(tot, round(tot * share0))) // 4) * 4
        s1 = -(-max(4, tot - s0) // 4) * 4
        return s0, s1

    def build_edges(s0, s1):
        smax = max(s0, s1)
        parts = []
        lo = 0
        for st in (s0, s1):
            cap = NS * st * CHUNK
            hi = min(e, lo + cap)
            padn = cap - (hi - lo)
            seg_s = jnp.concatenate(
                [edge_index[0, lo:hi], jnp.zeros((padn,), jnp.int32)])
            seg_d = jnp.concatenate(
                [edge_index[1, lo:hi],
                 n + jnp.arange(padn, dtype=jnp.int32) % (n_pad - n)])
            seg = jnp.stack([seg_s.reshape(NS, st, CHUNK),
                             seg_d.reshape(NS, st, CHUNK)], axis=2)
            if st < smax:
                seg = jnp.pad(seg, ((0, 0), (0, smax - st), (0, 0), (0, 0)))
            parts.append(seg)
            lo = hi
        return jnp.concatenate(parts, axis=0)

    s0_1, s1_1 = split_steps(0.353)
    s0_2, s1_2 = split_steps(0.40)
    edges1 = build_edges(s0_1, s1_1)
    edges2 = build_edges(s0_2, s1_2)
    zeros_f = jnp.zeros((rpt, f), jnp.float32)
    zeros_d2 = jnp.zeros((rpt, d2), jnp.float32)
    w2a_p = jnp.concatenate(
        [W2a, jnp.zeros((f, d2 - c_out), jnp.float32)], axis=1)
    b2a_p = jnp.concatenate(
        [b2a, jnp.zeros((d2 - c_out,), jnp.float32)]).reshape(1, d2)
    w2b_p = jnp.pad(W2b, ((0, d2 - c_out), (0, d2 - c_out)))
    b2b_p = jnp.concatenate(
        [b2b, jnp.full((d2 - c_out,), -1e30, jnp.float32)]).reshape(1, d2)
    b1a_r = b1a.reshape(1, f)
    b1b_r = b1b.reshape(1, f)

    agg1 = _make_sc_agg(n, f, n_pad, s0_1, s1_1)(x, edges1, zeros_f)

    g2 = pl.pallas_call(
        _tc1_body,
        grid=grid,
        in_specs=[
            pl.BlockSpec((blk, f), lambda i: (i, 0)),
            pl.BlockSpec((NC, blk, f), lambda i: (0, i, 0)),
            pl.BlockSpec((f, f), lambda i: (0, 0)),
            pl.BlockSpec((1, f), lambda i: (0, 0)),
            pl.BlockSpec((f, f), lambda i: (0, 0)),
            pl.BlockSpec((1, f), lambda i: (0, 0)),
            pl.BlockSpec((f, d2), lambda i: (0, 0)),
        ],
        out_specs=pl.BlockSpec((blk, d2), lambda i: (i, 0)),
        out_shape=jax.ShapeDtypeStruct((n, d2), jnp.float32),
    )(x, agg1, W1a, b1a_r, W1b, b1b_r, w2a_p)

    agg2 = _make_sc_agg(n, d2, n_pad, s0_2, s1_2)(g2, edges2, zeros_d2)

    out = pl.pallas_call(
        _tc2_body,
        grid=grid,
        in_specs=[
            pl.BlockSpec((blk, d2), lambda i: (i, 0)),
            pl.BlockSpec((NC, blk, d2), lambda i: (0, i, 0)),
            pl.BlockSpec((1, d2), lambda i: (0, 0)),
            pl.BlockSpec((d2, d2), lambda i: (0, 0)),
            pl.BlockSpec((1, d2), lambda i: (0, 0)),
        ],
        out_specs=pl.BlockSpec((blk, d2), lambda i: (i, 0)),
        out_shape=jax.ShapeDtypeStruct((n, d2), jnp.float32),
    )(g2, agg2, b2a_p, w2b_p, b2b_p)

    return out[:, :c_out]


# speedup vs baseline: 1.4955x; 1.4955x over previous
import functools

import jax
import jax.numpy as jnp
from jax import lax
from jax.experimental import pallas as pl
from jax.experimental.pallas import tpu as pltpu
from jax.experimental.pallas import tpu_sc as plsc

NC = 2
NS = 16
LANES = 16
CHUNK = 128


def _make_sc_agg(n_table, d, n_pad, steps):
    rpt = n_pad // NS
    mesh = plsc.VectorSubcoreMesh(core_axis_name="c", subcore_axis_name="s")

    @functools.partial(
        pl.kernel,
        mesh=mesh,
        compiler_params=pltpu.CompilerParams(use_tc_tiling_on_sc=(d % 128 == 0)),
        out_type=jax.ShapeDtypeStruct((NC, n_pad, d), jnp.float32),
        scratch_types=[
            pltpu.VMEM((steps, CHUNK), jnp.int32),
            pltpu.VMEM((steps, CHUNK), jnp.int32),
            pltpu.VMEM((CHUNK, d), jnp.float32),
            pltpu.VMEM_SHARED((n_pad, d), jnp.float32),
            pltpu.SemaphoreType.DMA,
        ],
    )
    def agg(table_hbm, src_hbm, dst_hbm, zeros_hbm, out_hbm,
            src_v, dst_v, rows_v, acc_sh, sem):
        c = lax.axis_index("c")
        s = lax.axis_index("s")
        tile = c * NS + s
        pltpu.sync_copy(zeros_hbm, acc_sh.at[pl.ds(s * rpt, rpt)])
        pltpu.sync_copy(src_hbm.at[tile], src_v)
        pltpu.sync_copy(dst_hbm.at[tile], dst_v)
        plsc.subcore_barrier()

        def body(i, carry):
            pltpu.async_copy(table_hbm.at[src_v.at[i]], rows_v, sem).wait()
            pltpu.sync_copy(rows_v, acc_sh.at[dst_v.at[i]], add=True)
            return carry

        lax.fori_loop(0, steps, body, 0)
        plsc.subcore_barrier()
        pltpu.sync_copy(acc_sh.at[pl.ds(s * rpt, rpt)],
                        out_hbm.at[c, pl.ds(s * rpt, rpt)])

    return agg


def _tc1_body(x_ref, agg_ref, w1a_ref, b1a_ref, w1b_ref, b1b_ref, w2a_ref,
              out_ref):
    z = x_ref[...] + agg_ref[0] + agg_ref[1]
    z1 = jnp.dot(z, w1a_ref[...], preferred_element_type=jnp.float32)
    z1 = jnp.maximum(z1 + b1a_ref[...], 0.0)
    h1 = jnp.dot(z1, w1b_ref[...], preferred_element_type=jnp.float32)
    h1 = h1 + b1b_ref[...]
    out_ref[...] = jnp.dot(h1, w2a_ref[...], preferred_element_type=jnp.float32)


def _tc2_body(g2_ref, agg_ref, b2a_ref, w2b_ref, b2b_ref, out_ref):
    z = g2_ref[...] + agg_ref[0] + agg_ref[1]
    z = jnp.maximum(z + b2a_ref[...], 0.0)
    h2 = jnp.dot(z, w2b_ref[...], preferred_element_type=jnp.float32)
    h2 = h2 + b2b_ref[...]
    m = jnp.max(h2, axis=1, keepdims=True)
    lse = jnp.log(jnp.sum(jnp.exp(h2 - m), axis=1, keepdims=True)) + m
    out_ref[...] = h2 - lse


def kernel(x, edge_index, W1a, b1a, W1b, b1b, W2a, b2a, W2b, b2b):
    n, f = x.shape
    e = edge_index.shape[1]
    c_out = W2b.shape[1]
    d2 = 48

    n_tiles = NC * NS
    steps = -(-e // (n_tiles * CHUNK))
    ep = n_tiles * CHUNK * steps
    blk = 1024
    grid = (-(-n // blk),)
    n_pad = grid[0] * blk

    src = jnp.concatenate(
        [edge_index[0], jnp.zeros((ep - e,), jnp.int32)]).reshape(
            n_tiles, steps, CHUNK)
    pad_dst = n + jnp.arange(ep - e, dtype=jnp.int32) % (n_pad - n)
    dst = jnp.concatenate([edge_index[1], pad_dst]).reshape(
        n_tiles, steps, CHUNK)
    zeros_f = jnp.zeros((n_pad // NS, f), jnp.float32)
    zeros_d2 = jnp.zeros((n_pad // NS, d2), jnp.float32)
    w2a_p = jnp.concatenate(
        [W2a, jnp.zeros((f, d2 - c_out), jnp.float32)], axis=1)
    b2a_p = jnp.concatenate(
        [b2a, jnp.zeros((d2 - c_out,), jnp.float32)]).reshape(1, d2)
    w2b_p = jnp.pad(W2b, ((0, d2 - c_out), (0, d2 - c_out)))
    b2b_p = jnp.concatenate(
        [b2b, jnp.full((d2 - c_out,), -1e30, jnp.float32)]).reshape(1, d2)
    b1a_r = b1a.reshape(1, f)
    b1b_r = b1b.reshape(1, f)

    agg1 = _make_sc_agg(n, f, n_pad, steps)(x, src, dst, zeros_f)

    g2 = pl.pallas_call(
        _tc1_body,
        grid=grid,
        in_specs=[
            pl.BlockSpec((blk, f), lambda i: (i, 0)),
            pl.BlockSpec((NC, blk, f), lambda i: (0, i, 0)),
            pl.BlockSpec((f, f), lambda i: (0, 0)),
            pl.BlockSpec((1, f), lambda i: (0, 0)),
            pl.BlockSpec((f, f), lambda i: (0, 0)),
            pl.BlockSpec((1, f), lambda i: (0, 0)),
            pl.BlockSpec((f, d2), lambda i: (0, 0)),
        ],
        out_specs=pl.BlockSpec((blk, d2), lambda i: (i, 0)),
        out_shape=jax.ShapeDtypeStruct((n, d2), jnp.float32),
    )(x, agg1, W1a, b1a_r, W1b, b1b_r, w2a_p)

    agg2 = _make_sc_agg(n, d2, n_pad, steps)(g2, src, dst, zeros_d2)

    out = pl.pallas_call(
        _tc2_body,
        grid=grid,
        in_specs=[
            pl.BlockSpec((blk, d2), lambda i: (i, 0)),
            pl.BlockSpec((NC, blk, d2), lambda i: (0, i, 0)),
            pl.BlockSpec((1, d2), lambda i: (0, 0)),
            pl.BlockSpec((d2, d2), lambda i: (0, 0)),
            pl.BlockSpec((1, d2), lambda i: (0, 0)),
        ],
        out_specs=pl.BlockSpec((blk, d2), lambda i: (i, 0)),
        out_shape=jax.ShapeDtypeStruct((n, d2), jnp.float32),
    )(g2, agg2, b2a_p, w2b_p, b2b_p)

    return out[:, :c_out]
